# baseline (speedup 1.0000x reference)
import functools

import jax
import jax.numpy as jnp
from jax import lax
from jax.experimental import pallas as pl
from jax.experimental.pallas import tpu as pltpu
from jax.experimental.pallas import tpu_sc as plsc

B = 16384
F = 26
NC, NS, L = 2, 16, 16
NW = NC * NS
BW = B // NW
NIDX = F * BW
CHUNK = 128
NCH = BW // CHUNK


LAG = 8


def _sc_body(xt_hbm, fc_hbm, off_hbm, bias_hbm, out_hbm,
             xv, rows, outv, offr, biasr, *sems):
    wid = lax.axis_index("s") * NC + lax.axis_index("c")
    base = wid * BW

    pltpu.sync_copy(off_hbm, offr.at[pl.ds(0, F)])
    pltpu.sync_copy(bias_hbm, biasr.at[pl.ds(0, 1)])
    gsems, isems = sems[:LAG], sems[LAG:]

    def xrow_copy(f):
        return pltpu.make_async_copy(
            xt_hbm.at[pl.ds(f, 1), pl.ds(base, BW)],
            xv.at[pl.ds(f, 1), :],
            isems[f % LAG],
        )
    for f in range(LAG):
        xrow_copy(f).start()

    _dnums = lax.GatherDimensionNumbers(
        offset_dims=(), collapsed_slice_dims=(0,), start_index_map=(0,))
    o_lo = offr[pl.ds(0, L)]
    o_hi = offr[pl.ds(L, L)]
    def _splat(vec, j):
        return lax.gather(
            vec, jnp.full((L, 1), j, jnp.int32), _dnums, (1,),
            mode=lax.GatherScatterMode.PROMISE_IN_BOUNDS)
    bias_vec = _splat(biasr[pl.ds(0, L)], 0)
    @pl.loop(0, BW // L)
    def _init(c):
        outv[pl.ds(c * L, L)] = bias_vec

    def fire(f):
        off_f = _splat(o_lo, f) if f < L else _splat(o_hi, f - L)
        xrow_copy(f).wait()
        if f + LAG < F:
            xrow_copy(f + LAG).start()
        for c4 in range(NCH):
            @pl.loop(c4 * (CHUNK // L), (c4 + 1) * (CHUNK // L))
            def _idx(c):
                xv[f, pl.ds(c * L, L)] = xv[f, pl.ds(c * L, L)] + off_f
            pltpu.make_async_copy(
                fc_hbm.at[xv.at[pl.ds(f, 1), pl.ds(c4 * CHUNK, CHUNK)]],
                rows.at[pl.ds(0, 1), pl.ds(f * BW + c4 * CHUNK, CHUNK)],
                gsems[f % LAG],
            ).start()

    def drain_acc(f):
        pltpu.make_async_copy(
            fc_hbm.at[pl.ds(0, 1), pl.ds(0, BW)],
            rows.at[pl.ds(0, 1), pl.ds(f * BW, BW)],
            gsems[f % LAG],
        ).wait()
        @pl.loop(0, BW // L)
        def _acc(c):
            outv[pl.ds(c * L, L)] = (
                outv[pl.ds(c * L, L)] + rows[0, pl.ds(f * BW + c * L, L)]
            )

    for f in range(LAG):
        fire(f)
    for f in range(LAG, F):
        drain_acc(f - LAG)
        fire(f)
    for f in range(F - LAG, F):
        drain_acc(f)

    pltpu.sync_copy(outv, out_hbm.at[pl.ds(base, BW)])


@jax.jit
def _features_linear(xt, fcr, offsets, bias):
    mesh = plsc.VectorSubcoreMesh(core_axis_name="c", subcore_axis_name="s")
    return pl.kernel(
        _sc_body,
        out_type=jax.ShapeDtypeStruct((B,), jnp.float32),
        mesh=mesh,
        scratch_types=[
            pltpu.VMEM((F, BW), jnp.int32),
            pltpu.VMEM((1, NIDX), jnp.float32),
            pltpu.VMEM((BW,), jnp.float32),
            pltpu.VMEM((2 * L,), jnp.int32),
            pltpu.VMEM((L,), jnp.float32),
        ] + [pltpu.SemaphoreType.DMA] * (2 * LAG),
    )(xt, fcr, offsets, bias)


def kernel(x, fc, bias, offsets):
    xt = x.T
    out = _features_linear(xt, fc.reshape(1, -1), offsets, bias)
    return out.reshape(B, 1)


# speedup vs baseline: 1.0125x; 1.0125x over previous
import functools

import jax
import jax.numpy as jnp
from jax import lax
from jax.experimental import pallas as pl
from jax.experimental.pallas import tpu as pltpu
from jax.experimental.pallas import tpu_sc as plsc

B = 16384
F = 26
NC, NS, L = 2, 16, 16
NW = NC * NS
BW = B // NW
NIDX = F * BW
CHUNK = 128
NCH = BW // CHUNK


LAG = 8


def _sc_body(xt_hbm, fc_hbm, off_hbm, bias_hbm, out_hbm,
             xv, rows, outv, offr, biasr, *sems):
    wid = lax.axis_index("s") * NC + lax.axis_index("c")
    base = wid * BW

    pltpu.sync_copy(xt_hbm.at[:, pl.ds(base, BW)], xv)
    pltpu.sync_copy(off_hbm, offr.at[pl.ds(0, F)])
    pltpu.sync_copy(bias_hbm, biasr.at[pl.ds(0, 1)])
    gsems = sems

    _dnums = lax.GatherDimensionNumbers(
        offset_dims=(), collapsed_slice_dims=(0,), start_index_map=(0,))
    o_lo = offr[pl.ds(0, L)]
    o_hi = offr[pl.ds(L, L)]
    def _splat(vec, j):
        return lax.gather(
            vec, jnp.full((L, 1), j, jnp.int32), _dnums, (1,),
            mode=lax.GatherScatterMode.PROMISE_IN_BOUNDS)
    bias_vec = _splat(biasr[pl.ds(0, L)], 0)
    @pl.loop(0, BW // L)
    def _init(c):
        outv[pl.ds(c * L, L)] = bias_vec

    def fire(f):
        off_f = _splat(o_lo, f) if f < L else _splat(o_hi, f - L)
        for c4 in range(NCH):
            @pl.loop(c4 * (CHUNK // L), (c4 + 1) * (CHUNK // L))
            def _idx(c):
                xv[f, pl.ds(c * L, L)] = xv[f, pl.ds(c * L, L)] + off_f
            pltpu.make_async_copy(
                fc_hbm.at[xv.at[pl.ds(f, 1), pl.ds(c4 * CHUNK, CHUNK)]],
                rows.at[pl.ds(0, 1), pl.ds(f * BW + c4 * CHUNK, CHUNK)],
                gsems[f % LAG],
            ).start()

    def drain_acc(f):
        pltpu.make_async_copy(
            fc_hbm.at[pl.ds(0, 1), pl.ds(0, BW)],
            rows.at[pl.ds(0, 1), pl.ds(f * BW, BW)],
            gsems[f % LAG],
        ).wait()
        @pl.loop(0, BW // L)
        def _acc(c):
            outv[pl.ds(c * L, L)] = (
                outv[pl.ds(c * L, L)] + rows[0, pl.ds(f * BW + c * L, L)]
            )

    for f in range(LAG):
        fire(f)
    for f in range(LAG, F):
        drain_acc(f - LAG)
        fire(f)
    for f in range(F - LAG, F):
        drain_acc(f)

    pltpu.sync_copy(outv, out_hbm.at[pl.ds(base, BW)])


@jax.jit
def _features_linear(xt, fcr, offsets, bias):
    mesh = plsc.VectorSubcoreMesh(core_axis_name="c", subcore_axis_name="s")
    return pl.kernel(
        _sc_body,
        out_type=jax.ShapeDtypeStruct((B,), jnp.float32),
        mesh=mesh,
        scratch_types=[
            pltpu.VMEM((F, BW), jnp.int32),
            pltpu.VMEM((1, NIDX), jnp.float32),
            pltpu.VMEM((BW,), jnp.float32),
            pltpu.VMEM((2 * L,), jnp.int32),
            pltpu.VMEM((L,), jnp.float32),
        ] + [pltpu.SemaphoreType.DMA] * LAG,
    )(xt, fcr, offsets, bias)


def kernel(x, fc, bias, offsets):
    xt = x.T
    out = _features_linear(xt, fc.reshape(1, -1), offsets, bias)
    return out.reshape(B, 1)
